# Initial kernel scaffold; baseline (speedup 1.0000x reference)
import jax
import jax.numpy as jnp
from jax.experimental import pallas as pl


def kernel(y, table):
    raise NotImplementedError("write your pallas kernel here")


# speedup vs baseline: 1.9645x; 1.9645x over previous
import jax
import jax.numpy as jnp
from jax import lax
from jax.experimental import pallas as pl
from jax.experimental.pallas import tpu as pltpu
from jax.experimental.pallas import tpu_sc as plsc

_K = 1000000
_M = 64
_B = 16384
_L = 50
_BL = _B * _L

_NC, _NS = 2, 16
_NW = _NC * _NS
_RPW = _BL // _NW
_CHUNK = 128
_NCHUNK = _RPW // _CHUNK
_NBUF = 4


def _scratch():
    out = [pltpu.VMEM((_NCHUNK, _CHUNK), jnp.int32)]
    out += [pltpu.VMEM((_CHUNK, _M), jnp.float32)] * _NBUF
    out += [pltpu.SemaphoreType.DMA] * (2 * _NBUF)
    return out


def _body(idx_hbm, table_hbm, out_hbm,
          idx_v, r0, r1, r2, r3,
          g0, g1, g2, g3, s0, s1, s2, s3):
    rows = (r0, r1, r2, r3)
    gsem = (g0, g1, g2, g3)
    ssem = (s0, s1, s2, s3)
    wid = lax.axis_index("s") * _NC + lax.axis_index("c")
    dst0 = wid * _RPW

    pltpu.sync_copy(idx_hbm.at[pl.ds(wid * _NCHUNK, _NCHUNK)], idx_v)

    def _wait_store(b):
        pltpu.make_async_copy(
            rows[b], out_hbm.at[pl.ds(0, _CHUNK)], ssem[b]).wait()

    def _gather(t, b):
        return pltpu.async_copy(
            table_hbm.at[idx_v.at[t]], rows[b], gsem[b])

    def _store(t, b):
        return pltpu.async_copy(
            rows[b], out_hbm.at[pl.ds(dst0 + t * _CHUNK, _CHUNK)], ssem[b])

    descs = [_gather(b, b) for b in range(_NBUF)]
    descs[0].wait()

    @pl.when(wid == 0)
    def _zero_head():
        z = jnp.zeros((16,), jnp.float32)
        for r in range(_L):
            for c in range(_M // 16):
                r0[r, pl.ds(c * 16, 16)] = z

    _store(0, 0)
    for b in range(1, _NBUF):
        descs[b].wait()
        _store(b, b)

    def _group(i):
        ds = []
        for b in range(_NBUF):
            _wait_store(b)
            ds.append(_gather(i + b, b))
        for b in range(_NBUF):
            ds[b].wait()
            _store(i + b, b)

    n_groups = _NCHUNK // _NBUF
    lax.fori_loop(1, n_groups - 1,
                  lambda k, c: (_group(k * _NBUF), c)[1], 0,
                  unroll=False)

    base = (n_groups - 1) * _NBUF
    for b in range(_NBUF):
        _wait_store(b)
        _gather(base + b, b).wait()
        _store(base + b, b).wait()


_emb_shift_kernel = pl.kernel(
    _body,
    out_type=jax.ShapeDtypeStruct((_BL, _M), jnp.float32),
    mesh=plsc.VectorSubcoreMesh(
        core_axis_name="c", subcore_axis_name="s",
        num_cores=_NC, num_subcores=_NS),
    scratch_types=_scratch(),
    compiler_params=pltpu.CompilerParams(use_tc_tiling_on_sc=False),
)


def kernel(y, table):
    yf = y.reshape(_BL).astype(jnp.int32)
    idx = jnp.pad(yf[:_BL - _L], (_L, 0))
    out = _emb_shift_kernel(idx.reshape(_BL // _CHUNK, _CHUNK), table)
    return out.reshape(_B, _L, _M)
